# baseline (speedup 1.0000x reference)
import functools

import jax
import jax.numpy as jnp
from jax import lax
from jax.experimental import pallas as pl
from jax.experimental.pallas import tpu as pltpu
from jax.experimental.pallas import tpu_sc as plsc

VOCAB = 100277
EMBED = 64
SEQ = 32
BATCH = 4096

N_FLAT = BATCH * SEQ
NW = 32
ROWS_PER_W = N_FLAT // NW
CH = 128
NCH = ROWS_PER_W // CH


def _sc_gather(table, x):
  mesh = plsc.VectorSubcoreMesh(core_axis_name="c", subcore_axis_name="s")

  @functools.partial(
      pl.kernel,
      mesh=mesh,
      compiler_params=pltpu.CompilerParams(use_tc_tiling_on_sc=True),
      out_type=jax.ShapeDtypeStruct((N_FLAT, 128), jnp.float32),
      scratch_types=[
          pltpu.VMEM((ROWS_PER_W,), jnp.float32),
          pltpu.VMEM((NCH, CH), jnp.int32),
          pltpu.VMEM((CH, 128), jnp.float32),
          pltpu.VMEM((CH, 128), jnp.float32),
          pltpu.SemaphoreType.DMA,
          pltpu.SemaphoreType.DMA,
      ],
  )
  def k(table_hbm, x_hbm, out_hbm, xv, idx_v, buf0, buf1, sem0, sem1):
    wid = lax.axis_index("s") * 2 + lax.axis_index("c")
    base = wid * ROWS_PER_W

    pltpu.sync_copy(x_hbm.at[pl.ds(base, ROWS_PER_W)], xv)

    @pl.loop(0, NCH)
    def _conv(j):
      for m in range(8):
        idx_v[j, pl.ds(m * 16, 16)] = xv[
            pl.ds(j * CH + m * 16, 16)].astype(jnp.int32)

    bufs = (buf0, buf1)
    sems = (sem0, sem1)

    pltpu.async_copy(table_hbm.at[idx_v.at[0]], buf0, sem0)

    @pl.loop(0, NCH, step=2)
    def _body(j):
      for b in range(2):
        cur = j + b

        @pl.when(cur + 1 < NCH)
        def _():
          pltpu.async_copy(
              table_hbm.at[idx_v.at[cur + 1]], bufs[1 - b], sems[1 - b])

        pltpu.make_async_copy(
            table_hbm.at[idx_v.at[cur]], bufs[b], sems[b]).wait()
        pltpu.sync_copy(bufs[b], out_hbm.at[pl.ds(base + cur * CH, CH)])

  return k(table, x)


def _tc_mlp(emb4, w1s, b1, w2t, b2, w3t, b3):
  BB = 512
  OUTP = w3t.shape[1]

  def body(e_ref, w1_ref, b1_ref, w2_ref, b2_ref, w3_ref, b3_ref, o_ref):
    h = jnp.dot(
        e_ref[:, 0, 0:EMBED], w1_ref[0], preferred_element_type=jnp.float32)
    for s in range(1, SEQ):
      h += jnp.dot(
          e_ref[:, s, 0:EMBED], w1_ref[s],
          preferred_element_type=jnp.float32)
    h = jnp.maximum(h + b1_ref[...], 0.0)
    h = jnp.dot(h, w2_ref[...], preferred_element_type=jnp.float32)
    h = jnp.maximum(h + b2_ref[...], 0.0)
    o_ref[...] = (
        jnp.dot(h, w3_ref[...], preferred_element_type=jnp.float32)
        + b3_ref[...])

  full = lambda a: pl.BlockSpec(a.shape, lambda i: (0,) * a.ndim)
  return pl.pallas_call(
      body,
      grid=(BATCH // BB,),
      in_specs=[
          pl.BlockSpec((BB, SEQ, 128), lambda i: (i, 0, 0)),
          full(w1s), full(b1), full(w2t), full(b2), full(w3t), full(b3),
      ],
      out_specs=pl.BlockSpec((BB, OUTP), lambda i: (i, 0)),
      out_shape=jax.ShapeDtypeStruct((BATCH, OUTP), jnp.float32),
  )(emb4, w1s, b1, w2t, b2, w3t, b3)


def kernel(x, table, W1, b1, W2, b2, W3, b3):
  table_p = jnp.pad(table, ((0, (-VOCAB) % 8), (0, 128 - EMBED)))
  x2 = x.astype(jnp.float32).reshape(N_FLAT)
  emb = _sc_gather(table_p, x2)
  emb4 = emb.reshape(BATCH, SEQ, 128)

  nout = W3.shape[0]
  w1s = W1.T.reshape(SEQ, EMBED, 128)
  w3t = jnp.zeros((W3.shape[1], 32), jnp.float32).at[:, :nout].set(W3.T)
  b3p = jnp.zeros((1, 32), jnp.float32).at[:, :nout].set(b3[None, :])
  out = _tc_mlp(emb4, w1s, b1[None, :], W2.T, b2[None, :], w3t, b3p)
  return out[:, :nout]


# speedup vs baseline: 1.2224x; 1.2224x over previous
import functools

import jax
import jax.numpy as jnp
from jax import lax
from jax.experimental import pallas as pl
from jax.experimental.pallas import tpu as pltpu
from jax.experimental.pallas import tpu_sc as plsc

VOCAB = 100277
EMBED = 64
SEQ = 32
BATCH = 4096

N_FLAT = BATCH * SEQ
N_PAIR = N_FLAT // 2
NW = 32
ROWS_PER_W = N_FLAT // NW
PAIRS_PER_W = N_PAIR // NW
CH = 128
NCH = ROWS_PER_W // CH


NCHP = PAIRS_PER_W // CH
BPW = BATCH // NW
SEQH = SEQ // 2


def _sc_gather(table, x):
  mesh = plsc.VectorSubcoreMesh(core_axis_name="c", subcore_axis_name="s")

  @functools.partial(
      pl.kernel,
      mesh=mesh,
      compiler_params=pltpu.CompilerParams(use_tc_tiling_on_sc=False),
      out_type=jax.ShapeDtypeStruct((N_PAIR, 128), jnp.float32),
      scratch_types=[
          pltpu.VMEM((BPW * SEQ,), jnp.int32),
          pltpu.VMEM((NCHP, CH), jnp.int32),
          pltpu.VMEM((NCHP, CH), jnp.int32),
          pltpu.VMEM((CH, 128), jnp.float32),
          pltpu.VMEM((CH, 128), jnp.float32),
          pltpu.VMEM((CH, 128), jnp.float32),
          pltpu.VMEM((CH, 128), jnp.float32),
          pltpu.SemaphoreType.DMA,
          pltpu.SemaphoreType.DMA,
      ],
  )
  def k(table_hbm, x_hbm, out_hbm, xv, idxl, idxr,
        bufe0, bufo0, bufe1, bufo1, sem0, sem1):
    wid = lax.axis_index("s") * 2 + lax.axis_index("c")
    base = wid * PAIRS_PER_W

    pltpu.sync_copy(x_hbm.at[pl.ds(wid * BPW * SEQ, BPW * SEQ)], xv)

    @pl.loop(0, NCHP)
    def _deint(j):
      for m in range(8):
        off = (j * 8 + m) * SEQ
        idxl[j, pl.ds(m * 16, 16)] = xv[pl.ds(off, SEQH)]
        idxr[j, pl.ds(m * 16, 16)] = xv[pl.ds(off + SEQH, SEQH)]

    bufs = ((bufe0, bufo0), (bufe1, bufo1))
    sems = (sem0, sem1)

    def start(cur, b):
      be, bo = bufs[b]
      pltpu.async_copy(table_hbm.at[idxl.at[cur]], be, sems[b])
      pltpu.async_copy(table_hbm.at[idxr.at[cur]], bo, sems[b])

    def finish(cur, b):
      be, bo = bufs[b]
      pltpu.make_async_copy(table_hbm.at[idxl.at[cur]], be, sems[b]).wait()
      pltpu.make_async_copy(table_hbm.at[idxr.at[cur]], bo, sems[b]).wait()
      rows = out_hbm.at[pl.ds(base + cur * CH, CH)]
      pltpu.sync_copy(be.at[:, pl.ds(0, EMBED)], rows.at[:, pl.ds(0, EMBED)])
      pltpu.sync_copy(bo.at[:, pl.ds(0, EMBED)],
                      rows.at[:, pl.ds(EMBED, EMBED)])

    start(0, 0)

    @pl.loop(0, NCHP, step=2)
    def _body(j):
      for b in range(2):
        cur = j + b

        @pl.when(cur + 1 < NCHP)
        def _():
          start(cur + 1, 1 - b)

        finish(cur, b)

  return k(table, x)


NP = SEQ * EMBED // 128


def _tc_mlp(emb3, w1r, b1, w2t, b2, w3t, b3):
  BB = 512
  OUTP = w3t.shape[1]

  def body(e_ref, w1_ref, b1_ref, w2_ref, b2_ref, w3_ref, b3_ref, o_ref):
    h = jnp.dot(
        e_ref[:, 0, :], w1_ref[0], preferred_element_type=jnp.float32)
    for p in range(1, NP):
      h += jnp.dot(
          e_ref[:, p, :], w1_ref[p], preferred_element_type=jnp.float32)
    h = jnp.maximum(h + b1_ref[...], 0.0)
    h = jnp.dot(h, w2_ref[...], preferred_element_type=jnp.float32)
    h = jnp.maximum(h + b2_ref[...], 0.0)
    o_ref[...] = (
        jnp.dot(h, w3_ref[...], preferred_element_type=jnp.float32)
        + b3_ref[...])

  full = lambda a: pl.BlockSpec(a.shape, lambda i: (0,) * a.ndim)
  return pl.pallas_call(
      body,
      grid=(BATCH // BB,),
      in_specs=[
          pl.BlockSpec((BB, NP, 128), lambda i: (i, 0, 0)),
          full(w1r), full(b1), full(w2t), full(b2), full(w3t), full(b3),
      ],
      out_specs=pl.BlockSpec((BB, OUTP), lambda i: (i, 0)),
      out_shape=jax.ShapeDtypeStruct((BATCH, OUTP), jnp.float32),
  )(emb3, w1r, b1, w2t, b2, w3t, b3)


def kernel(x, table, W1, b1, W2, b2, W3, b3):
  table_p = jnp.pad(table, ((0, (-VOCAB) % 8), (0, 128 - EMBED)))
  x2 = x.astype(jnp.int32).reshape(BATCH * SEQ)
  emb2 = _sc_gather(table_p, x2)
  emb3 = emb2.reshape(BATCH, NP, 128)

  nout = W3.shape[0]
  w1s = W1.T.reshape(SEQ, EMBED, 128)
  w1r = jnp.concatenate([w1s[:SEQH], w1s[SEQH:]], axis=1)
  w3t = jnp.zeros((W3.shape[1], 32), jnp.float32).at[:, :nout].set(W3.T)
  b3p = jnp.zeros((1, 32), jnp.float32).at[:, :nout].set(b3[None, :])
  out = _tc_mlp(emb3, w1r, b1[None, :], W2.T, b2[None, :], w3t, b3p)
  return out[:, :nout]


# speedup vs baseline: 1.2230x; 1.0005x over previous
import functools

import jax
import jax.numpy as jnp
from jax import lax
from jax.experimental import pallas as pl
from jax.experimental.pallas import tpu as pltpu
from jax.experimental.pallas import tpu_sc as plsc

VOCAB = 100277
EMBED = 64
SEQ = 32
BATCH = 4096

N_FLAT = BATCH * SEQ
N_PAIR = N_FLAT // 2
NW = 32
ROWS_PER_W = N_FLAT // NW
PAIRS_PER_W = N_PAIR // NW
CH = 128
NCH = ROWS_PER_W // CH


NCHP = PAIRS_PER_W // CH
BPW = BATCH // NW
SEQH = SEQ // 2


def _sc_gather(table, x, nbatch):
  mesh = plsc.VectorSubcoreMesh(core_axis_name="c", subcore_axis_name="s")

  npair = nbatch * SEQ // 2
  pairs_per_w = npair // NW
  nchp = pairs_per_w // CH
  bpw = nbatch // NW

  @functools.partial(
      pl.kernel,
      mesh=mesh,
      compiler_params=pltpu.CompilerParams(use_tc_tiling_on_sc=False),
      out_type=jax.ShapeDtypeStruct((npair, 128), jnp.float32),
      scratch_types=[
          pltpu.VMEM((bpw * SEQ,), jnp.int32),
          pltpu.VMEM((nchp, CH), jnp.int32),
          pltpu.VMEM((nchp, CH), jnp.int32),
          pltpu.VMEM((CH, 128), jnp.float32),
          pltpu.VMEM((CH, 128), jnp.float32),
          pltpu.VMEM((CH, 128), jnp.float32),
          pltpu.VMEM((CH, 128), jnp.float32),
          pltpu.SemaphoreType.DMA,
          pltpu.SemaphoreType.DMA,
      ],
  )
  def k(table_hbm, x_hbm, out_hbm, xv, idxl, idxr,
        bufe0, bufo0, bufe1, bufo1, sem0, sem1):
    wid = lax.axis_index("s") * 2 + lax.axis_index("c")
    base = wid * pairs_per_w

    pltpu.sync_copy(x_hbm.at[pl.ds(wid * bpw * SEQ, bpw * SEQ)], xv)

    @pl.loop(0, nchp)
    def _deint(j):
      for m in range(8):
        off = (j * 8 + m) * SEQ
        idxl[j, pl.ds(m * 16, 16)] = xv[pl.ds(off, SEQH)]
        idxr[j, pl.ds(m * 16, 16)] = xv[pl.ds(off + SEQH, SEQH)]

    bufs = ((bufe0, bufo0), (bufe1, bufo1))
    sems = (sem0, sem1)

    def start(cur, b):
      be, bo = bufs[b]
      pltpu.async_copy(table_hbm.at[idxl.at[cur]], be, sems[b])
      pltpu.async_copy(table_hbm.at[idxr.at[cur]], bo, sems[b])

    def finish(cur, b):
      be, bo = bufs[b]
      pltpu.make_async_copy(table_hbm.at[idxl.at[cur]], be, sems[b]).wait()
      pltpu.make_async_copy(table_hbm.at[idxr.at[cur]], bo, sems[b]).wait()
      rows = out_hbm.at[pl.ds(base + cur * CH, CH)]
      pltpu.sync_copy(be.at[:, pl.ds(0, EMBED)], rows.at[:, pl.ds(0, EMBED)])
      pltpu.sync_copy(bo.at[:, pl.ds(0, EMBED)],
                      rows.at[:, pl.ds(EMBED, EMBED)])

    start(0, 0)

    @pl.loop(0, nchp, step=2)
    def _body(j):
      for b in range(2):
        cur = j + b

        @pl.when(cur + 1 < nchp)
        def _():
          start(cur + 1, 1 - b)

        finish(cur, b)

  return k(table, x)


NP = SEQ * EMBED // 128


def _tc_mlp(emb3, w1r, b1, w2t, b2, w3t, b3):
  BB = 512
  nbatch = emb3.shape[0]
  OUTP = w3t.shape[1]

  def body(e_ref, w1_ref, b1_ref, w2_ref, b2_ref, w3_ref, b3_ref, o_ref):
    h = jnp.dot(
        e_ref[:, 0, :], w1_ref[0], preferred_element_type=jnp.float32)
    for p in range(1, NP):
      h += jnp.dot(
          e_ref[:, p, :], w1_ref[p], preferred_element_type=jnp.float32)
    h = jnp.maximum(h + b1_ref[...], 0.0)
    h = jnp.dot(h, w2_ref[...], preferred_element_type=jnp.float32)
    h = jnp.maximum(h + b2_ref[...], 0.0)
    o_ref[...] = (
        jnp.dot(h, w3_ref[...], preferred_element_type=jnp.float32)
        + b3_ref[...])

  full = lambda a: pl.BlockSpec(a.shape, lambda i: (0,) * a.ndim)
  return pl.pallas_call(
      body,
      grid=(nbatch // BB,),
      in_specs=[
          pl.BlockSpec((BB, NP, 128), lambda i: (i, 0, 0)),
          full(w1r), full(b1), full(w2t), full(b2), full(w3t), full(b3),
      ],
      out_specs=pl.BlockSpec((BB, OUTP), lambda i: (i, 0)),
      out_shape=jax.ShapeDtypeStruct((nbatch, OUTP), jnp.float32),
  )(emb3, w1r, b1, w2t, b2, w3t, b3)


def kernel(x, table, W1, b1, W2, b2, W3, b3):
  table_p = jnp.pad(table, ((0, (-VOCAB) % 8), (0, 128 - EMBED)))
  x2 = x.astype(jnp.int32).reshape(BATCH * SEQ)

  nout = W3.shape[0]
  w1s = W1.T.reshape(SEQ, EMBED, 128)
  w1r = jnp.concatenate([w1s[:SEQH], w1s[SEQH:]], axis=1)
  w3t = jnp.zeros((W3.shape[1], 32), jnp.float32).at[:, :nout].set(W3.T)
  b3p = jnp.zeros((1, 32), jnp.float32).at[:, :nout].set(b3[None, :])

  NS = 2
  nb = BATCH // NS
  outs = []
  for i in range(NS):
    emb2 = _sc_gather(table_p, lax.dynamic_slice(x2, (i * nb * SEQ,),
                                                 (nb * SEQ,)), nb)
    emb3 = emb2.reshape(nb, NP, 128)
    outs.append(
        _tc_mlp(emb3, w1r, b1[None, :], W2.T, b2[None, :], w3t, b3p))
  out = jnp.concatenate(outs, axis=0)
  return out[:, :nout]
